# baseline (speedup 1.0000x reference)
import jax
import jax.numpy as jnp
from jax.experimental import pallas as pl
from jax.experimental.pallas import tpu as pltpu

B, N, C, IN, HID, ATOMS = 2, 4096, 1024, 128, 64, 64
BLK = 512
NBLK = N // BLK
SCALE = HID ** -0.5


def _mmt(a, w):
    return jax.lax.dot_general(a, w, (((1,), (1,)), ((), ())),
                               preferred_element_type=jnp.float32)


def _ln(x, g, b, eps=1e-5):
    m = jnp.mean(x, axis=1, keepdims=True)
    xc = x - m
    v = jnp.mean(xc * xc, axis=1, keepdims=True)
    return xc / jnp.sqrt(v + eps) * g + b


def _fused(x_ref, adj_ref, inc_ref, imp_ref,
           nm_w, nm_b, cm_w, cm_b, atoms,
           q_w, q_b, k_w, k_b, s_w, s_b,
           c1_w, c1_b, c2_w, c2_b,
           pc_g, pc_b, f_w, f_b, n_g, n_b,
           out_ref, ent_ref,
           inc_s, h_s, cell_s, p_s):
    p = pl.program_id(0)
    i = pl.program_id(1)

    @pl.when(p == 0)
    def _phase0():
        @pl.when(i == 0)
        def _():
            cell_s[...] = jnp.zeros_like(cell_s)

        gate = jax.nn.sigmoid(imp_ref[...])
        inc_blk = inc_ref[...]
        inc_s[pl.ds(i * BLK, BLK), :] = inc_blk
        hs, cs = [], []
        for b in range(B):
            xg = x_ref[b] * gate
            hs.append(_mmt(xg, nm_w[...]) + nm_b[...])
            cs.append(jax.lax.dot_general(
                inc_blk, xg, (((0,), (0,)), ((), ())),
                preferred_element_type=jnp.float32))
        h_s[pl.ds(i * BLK, BLK), :] = jnp.concatenate(hs, axis=1)
        cell_s[...] += jnp.concatenate(cs, axis=1)

    @pl.when(p == 1)
    def _phase1():
        @pl.when(i == 0)
        def _cell_stage():
            kk = _mmt(atoms[...], k_w[...]) + k_b[...]
            ent = jnp.float32(0.0)
            for b in range(B):
                cell_b = cell_s[:, b * IN:(b + 1) * IN]
                h2 = _mmt(cell_b, cm_w[...]) + cm_b[...]
                q = _mmt(h2, q_w[...]) + q_b[...]
                attn = _mmt(q, kk) * SCALE
                m = jnp.max(attn, axis=1, keepdims=True)
                e = jnp.exp(attn - m)
                w = e / jnp.sum(e, axis=1, keepdims=True)
                pc = jnp.dot(w, atoms[...],
                             preferred_element_type=jnp.float32)
                p_s[:, b * HID:(b + 1) * HID] = pc
                ent = ent - jnp.sum(w * jnp.log(w + 1e-6))
            ent_ref[...] = jnp.reshape(ent / (B * C), (1, 1))

        agg = jnp.dot(adj_ref[...], h_s[...],
                      preferred_element_type=jnp.float32)
        pn = jnp.dot(inc_s[pl.ds(i * BLK, BLK), :], p_s[...],
                     preferred_element_type=jnp.float32)
        for b in range(B):
            ha = agg[:, b * HID:(b + 1) * HID]
            pnb = pn[:, b * HID:(b + 1) * HID]
            sur = ha - pnb
            err = jnp.sqrt(jnp.sum(sur * sur, axis=1, keepdims=True))
            conf = 1.0 / (1.0 + err)
            ps = _mmt(sur, s_w[...]) + s_b[...]
            r = jnp.maximum(_mmt(jnp.abs(sur), c1_w[...]) + c1_b[...], 0.0)
            lc = jax.nn.sigmoid(
                jnp.sum(r * c2_w[...], axis=1, keepdims=True) + c2_b[...])
            ge = ps * (conf * lc)
            processed = _ln(ge + ha, pc_g[...], pc_b[...])
            comb = jnp.concatenate([processed, pnb], axis=1)
            o = _mmt(comb, f_w[...]) + f_b[...]
            out_ref[b] = _ln(o, n_g[...], n_b[...])


def kernel(x_nodes, adjacency, incidence, node_importance, nm_w, nm_b, cm_w,
           cm_b, atoms, q_w, q_b, k_w, k_b, s_w, s_b, c1_w, c1_b, c2_w, c2_b,
           pc_g, pc_b, f_w, f_b, n_g, n_b):
    f32 = jnp.float32
    row = lambda v: jnp.reshape(v, (1, -1))
    imp = jnp.reshape(node_importance, (N, 1))

    def full(a):
        return pl.BlockSpec(a.shape, lambda p, i: (0,) * a.ndim)

    last = NBLK - 1
    in_specs = [
        pl.BlockSpec((B, BLK, IN), lambda p, i: (0, (1 - p) * i + p * last, 0)),
        pl.BlockSpec((BLK, N), lambda p, i: (p * i, 0)),
        pl.BlockSpec((BLK, C), lambda p, i: ((1 - p) * i + p * last, 0)),
        pl.BlockSpec((BLK, 1), lambda p, i: ((1 - p) * i + p * last, 0)),
    ]
    smalls = [nm_w, row(nm_b), cm_w, row(cm_b), atoms,
              q_w, row(q_b), k_w, row(k_b), s_w, row(s_b),
              c1_w, row(c1_b), c2_w, row(c2_b),
              row(pc_g), row(pc_b), f_w, row(f_b), row(n_g), row(n_b)]
    in_specs += [full(a) for a in smalls]

    out, ent = pl.pallas_call(
        _fused,
        grid=(2, NBLK),
        in_specs=in_specs,
        out_specs=[
            pl.BlockSpec((B, BLK, HID), lambda p, i: (0, p * i, 0)),
            pl.BlockSpec((1, 1), lambda p, i: (0, 0)),
        ],
        out_shape=[
            jax.ShapeDtypeStruct((B, N, HID), f32),
            jax.ShapeDtypeStruct((1, 1), f32),
        ],
        scratch_shapes=[
            pltpu.VMEM((N, C), f32),
            pltpu.VMEM((N, B * HID), f32),
            pltpu.VMEM((C, B * IN), f32),
            pltpu.VMEM((C, B * HID), f32),
        ],
        compiler_params=pltpu.CompilerParams(
            dimension_semantics=("arbitrary", "arbitrary")),
    )(x_nodes, adjacency, incidence, imp, *smalls)
    return out, ent[0, 0]


# speedup vs baseline: 1.0011x; 1.0011x over previous
import jax
import jax.numpy as jnp
from jax.experimental import pallas as pl
from jax.experimental.pallas import tpu as pltpu

B, N, C, IN, HID, ATOMS = 2, 4096, 1024, 128, 64, 64
BLK = 512
NBLK = N // BLK
SCALE = HID ** -0.5


def _mmt(a, w):
    return jax.lax.dot_general(a, w, (((1,), (1,)), ((), ())),
                               preferred_element_type=jnp.float32)


def _ln(x, g, b, eps=1e-5):
    m = jnp.mean(x, axis=1, keepdims=True)
    xc = x - m
    v = jnp.mean(xc * xc, axis=1, keepdims=True)
    return xc / jnp.sqrt(v + eps) * g + b


def _fused(x_ref, adj_ref, inc_ref, imp_ref,
           nm_w, nm_b, cm_w, cm_b, atoms,
           q_w, q_b, k_w, k_b, s_w, s_b,
           c1_w, c1_b, c2_w, c2_b,
           pc_g, pc_b, f_w, f_b, n_g, n_b,
           out_ref, ent_ref,
           inc_s, h_s, cell_s, p_s):
    p = pl.program_id(0)
    i = pl.program_id(1)

    @pl.when(p == 0)
    def _phase0():
        @pl.when(i == 0)
        def _():
            cell_s[...] = jnp.zeros_like(cell_s)

        gate = jax.nn.sigmoid(imp_ref[...])
        inc_blk = inc_ref[...].astype(jnp.bfloat16)
        inc_s[pl.ds(i * BLK, BLK), :] = inc_blk
        hs, cs = [], []
        for b in range(B):
            xg = x_ref[b] * gate
            xg16 = xg.astype(jnp.bfloat16)
            hs.append(_mmt(xg, nm_w[...]) + nm_b[...])
            cs.append(jax.lax.dot_general(
                inc_blk, xg16, (((0,), (0,)), ((), ())),
                preferred_element_type=jnp.float32))
        h_s[pl.ds(i * BLK, BLK), :] = jnp.concatenate(hs, axis=1).astype(jnp.bfloat16)
        cell_s[...] += jnp.concatenate(cs, axis=1)

    @pl.when(p == 1)
    def _phase1():
        @pl.when(i == 0)
        def _cell_stage():
            kk = _mmt(atoms[...], k_w[...]) + k_b[...]
            ent = jnp.float32(0.0)
            for b in range(B):
                cell_b = cell_s[:, b * IN:(b + 1) * IN]
                h2 = _mmt(cell_b, cm_w[...]) + cm_b[...]
                q = _mmt(h2, q_w[...]) + q_b[...]
                attn = _mmt(q, kk) * SCALE
                m = jnp.max(attn, axis=1, keepdims=True)
                e = jnp.exp(attn - m)
                w = e / jnp.sum(e, axis=1, keepdims=True)
                pc = jnp.dot(w, atoms[...],
                             preferred_element_type=jnp.float32)
                p_s[:, b * HID:(b + 1) * HID] = pc.astype(jnp.bfloat16)
                ent = ent - jnp.sum(w * jnp.log(w + 1e-6))
            ent_ref[...] = jnp.reshape(ent / (B * C), (1, 1))

        agg = jnp.dot(adj_ref[...].astype(jnp.bfloat16), h_s[...],
                      preferred_element_type=jnp.float32)
        pn = jnp.dot(inc_s[pl.ds(i * BLK, BLK), :], p_s[...],
                     preferred_element_type=jnp.float32)
        for b in range(B):
            ha = agg[:, b * HID:(b + 1) * HID]
            pnb = pn[:, b * HID:(b + 1) * HID]
            sur = ha - pnb
            err = jnp.sqrt(jnp.sum(sur * sur, axis=1, keepdims=True))
            conf = 1.0 / (1.0 + err)
            ps = _mmt(sur, s_w[...]) + s_b[...]
            r = jnp.maximum(_mmt(jnp.abs(sur), c1_w[...]) + c1_b[...], 0.0)
            lc = jax.nn.sigmoid(
                jnp.sum(r * c2_w[...], axis=1, keepdims=True) + c2_b[...])
            ge = ps * (conf * lc)
            processed = _ln(ge + ha, pc_g[...], pc_b[...])
            comb = jnp.concatenate([processed, pnb], axis=1)
            o = _mmt(comb, f_w[...]) + f_b[...]
            out_ref[b] = _ln(o, n_g[...], n_b[...])


def kernel(x_nodes, adjacency, incidence, node_importance, nm_w, nm_b, cm_w,
           cm_b, atoms, q_w, q_b, k_w, k_b, s_w, s_b, c1_w, c1_b, c2_w, c2_b,
           pc_g, pc_b, f_w, f_b, n_g, n_b):
    f32 = jnp.float32
    row = lambda v: jnp.reshape(v, (1, -1))
    imp = jnp.reshape(node_importance, (N, 1))

    def full(a):
        return pl.BlockSpec(a.shape, lambda p, i: (0,) * a.ndim)

    last = NBLK - 1
    in_specs = [
        pl.BlockSpec((B, BLK, IN), lambda p, i: (0, (1 - p) * i + p * last, 0)),
        pl.BlockSpec((BLK, N), lambda p, i: (p * i, 0)),
        pl.BlockSpec((BLK, C), lambda p, i: ((1 - p) * i + p * last, 0)),
        pl.BlockSpec((BLK, 1), lambda p, i: ((1 - p) * i + p * last, 0)),
    ]
    smalls = [nm_w, row(nm_b), cm_w, row(cm_b), atoms,
              q_w, row(q_b), k_w, row(k_b), s_w, row(s_b),
              c1_w, row(c1_b), c2_w, row(c2_b),
              row(pc_g), row(pc_b), f_w, row(f_b), row(n_g), row(n_b)]
    in_specs += [full(a) for a in smalls]

    out, ent = pl.pallas_call(
        _fused,
        grid=(2, NBLK),
        in_specs=in_specs,
        out_specs=[
            pl.BlockSpec((B, BLK, HID), lambda p, i: (0, p * i, 0)),
            pl.BlockSpec((1, 1), lambda p, i: (0, 0)),
        ],
        out_shape=[
            jax.ShapeDtypeStruct((B, N, HID), f32),
            jax.ShapeDtypeStruct((1, 1), f32),
        ],
        scratch_shapes=[
            pltpu.VMEM((N, C), jnp.bfloat16),
            pltpu.VMEM((N, B * HID), jnp.bfloat16),
            pltpu.VMEM((C, B * IN), f32),
            pltpu.VMEM((C, B * HID), jnp.bfloat16),
        ],
        compiler_params=pltpu.CompilerParams(
            dimension_semantics=("arbitrary", "arbitrary")),
    )(x_nodes, adjacency, incidence, imp, *smalls)
    return out, ent[0, 0]
